# baseline (speedup 1.0000x reference)
import functools

import jax
import jax.numpy as jnp
from jax import lax
from jax.experimental import pallas as pl
from jax.experimental.pallas import tpu as pltpu


def _nearest_idx_kernel(x_ref, w_ref, o_ref, mm_ref, bdotb_ref,
                        *, n_codes, bm, chunk, ktile):
    @pl.when(pl.program_id(0) == 0)
    def _init():
        w0 = w_ref[...]
        bdotb_ref[...] = jnp.sum(w0 * w0, axis=1, keepdims=True)

    x = x_ref[...]
    xs = -(x + x)
    mm_ref[...] = lax.dot_general(
        w_ref[...], xs, dimension_numbers=(((1,), (1,)), ((), ())),
        preferred_element_type=jnp.float32)
    xt = jnp.transpose(x)
    adota = jnp.sum(xt * xt, axis=0, keepdims=True)

    iotac = lax.broadcasted_iota(jnp.int32, (ktile, 1), 0).astype(jnp.float32)
    bdotb = bdotb_ref[...]

    for c in range(bm // chunk):
        lo, hi = c * chunk, (c + 1) * chunk
        adota_c = adota[:, lo:hi]
        val = jnp.full((1, chunk), jnp.inf, jnp.float32)
        idx = jnp.zeros((1, chunk), jnp.float32)
        for t in range(n_codes // ktile):
            tile = mm_ref[t * ktile:(t + 1) * ktile, lo:hi]
            btile = bdotb[t * ktile:(t + 1) * ktile, :]
            d = (adota_c + tile) + btile
            mt = jnp.min(d, axis=0, keepdims=True)
            loc = jnp.min(
                jnp.where(d == mt, iotac, float(n_codes)),
                axis=0, keepdims=True)
            locg = loc + float(t * ktile)
            upd = mt < val
            val = jnp.where(upd, mt, val)
            idx = jnp.where(upd, locg, idx)
        o_ref[0, 0, lo:hi] = idx[0].astype(jnp.int32)


def kernel(X, weight):
    B, S, D = X.shape
    K = weight.shape[0]
    M = B * S
    x2 = X.reshape(M, D)

    BM = 9216
    nblk = M // BM

    out = pl.pallas_call(
        functools.partial(_nearest_idx_kernel, n_codes=K, bm=BM,
                          chunk=256, ktile=128),
        grid=(nblk,),
        in_specs=[
            pl.BlockSpec((BM, D), lambda i: (i, 0)),
            pl.BlockSpec((K, D), lambda i: (0, 0)),
        ],
        out_specs=pl.BlockSpec((1, 1, BM), lambda i: (i, 0, 0)),
        out_shape=jax.ShapeDtypeStruct((nblk, 1, BM), jnp.int32),
        scratch_shapes=[
            pltpu.VMEM((K, BM), jnp.float32),
            pltpu.VMEM((K, 1), jnp.float32),
        ],
        compiler_params=pltpu.CompilerParams(
            dimension_semantics=("arbitrary",)),
    )(x2, weight)
    return out.reshape(B, S)


# speedup vs baseline: 1.0265x; 1.0265x over previous
import functools

import jax
import jax.numpy as jnp
from jax import lax
from jax.experimental import pallas as pl
from jax.experimental.pallas import tpu as pltpu


def _nearest_idx_kernel(x_ref, w_ref, o_ref, mm_ref, bdotb_ref,
                        *, n_codes, bm, chunk, ktile):
    @pl.when(pl.program_id(0) == 0)
    def _init():
        w0 = w_ref[...]
        bdotb_ref[...] = jnp.sum(w0 * w0, axis=1, keepdims=True)

    x = x_ref[...]
    xs = -(x + x)
    mm_ref[...] = lax.dot_general(
        w_ref[...], xs, dimension_numbers=(((1,), (1,)), ((), ())),
        preferred_element_type=jnp.float32)
    xt = jnp.transpose(x)
    adota = jnp.sum(xt * xt, axis=0, keepdims=True)

    iotac = lax.broadcasted_iota(jnp.int32, (ktile, 1), 0).astype(jnp.float32)
    bdotb = bdotb_ref[...]

    for c in range(bm // chunk):
        lo, hi = c * chunk, (c + 1) * chunk
        adota_c = adota[:, lo:hi]
        val = jnp.full((1, chunk), jnp.inf, jnp.float32)
        idx = jnp.zeros((1, chunk), jnp.float32)
        for t in range(n_codes // ktile):
            tile = mm_ref[t * ktile:(t + 1) * ktile, lo:hi]
            btile = bdotb[t * ktile:(t + 1) * ktile, :]
            d = (adota_c + tile) + btile
            mt = jnp.min(d, axis=0, keepdims=True)
            loc = jnp.min(
                jnp.where(d == mt, iotac, float(n_codes)),
                axis=0, keepdims=True)
            locg = loc + float(t * ktile)
            upd = mt < val
            val = jnp.where(upd, mt, val)
            idx = jnp.where(upd, locg, idx)
        o_ref[0, 0, lo:hi] = idx[0].astype(jnp.int32)


def kernel(X, weight):
    B, S, D = X.shape
    K = weight.shape[0]
    M = B * S
    x2 = X.reshape(M, D)

    BM = 4608
    nblk = M // BM

    out = pl.pallas_call(
        functools.partial(_nearest_idx_kernel, n_codes=K, bm=BM,
                          chunk=512, ktile=128),
        grid=(nblk,),
        in_specs=[
            pl.BlockSpec((BM, D), lambda i: (i, 0)),
            pl.BlockSpec((K, D), lambda i: (0, 0)),
        ],
        out_specs=pl.BlockSpec((1, 1, BM), lambda i: (i, 0, 0)),
        out_shape=jax.ShapeDtypeStruct((nblk, 1, BM), jnp.int32),
        scratch_shapes=[
            pltpu.VMEM((K, BM), jnp.float32),
            pltpu.VMEM((K, 1), jnp.float32),
        ],
        compiler_params=pltpu.CompilerParams(
            dimension_semantics=("arbitrary",)),
    )(x2, weight)
    return out.reshape(B, S)


# speedup vs baseline: 1.0270x; 1.0005x over previous
import functools

import jax
import jax.numpy as jnp
from jax import lax
from jax.experimental import pallas as pl
from jax.experimental.pallas import tpu as pltpu


def _nearest_idx_kernel(x_ref, w_ref, o_ref, bdotb_ref, *mm_refs,
                        n_codes, bm, chunk, ktile):
    @pl.when(pl.program_id(0) == 0)
    def _init():
        w0 = w_ref[...]
        bdotb_ref[...] = jnp.sum(w0 * w0, axis=1, keepdims=True)

    x = x_ref[...]
    xs = -(x + x)
    w = w_ref[...]
    for c in range(bm // chunk):
        lo, hi = c * chunk, (c + 1) * chunk
        mm_refs[c][...] = lax.dot_general(
            w, xs[lo:hi, :], dimension_numbers=(((1,), (1,)), ((), ())),
            preferred_element_type=jnp.float32)

    xt = jnp.transpose(x)
    adota = jnp.sum(xt * xt, axis=0, keepdims=True)

    iotac = lax.broadcasted_iota(jnp.int32, (ktile, 1), 0).astype(jnp.float32)
    bdotb = bdotb_ref[...]

    for c in range(bm // chunk):
        lo, hi = c * chunk, (c + 1) * chunk
        adota_c = adota[:, lo:hi]
        val = jnp.full((1, chunk), jnp.inf, jnp.float32)
        idx = jnp.zeros((1, chunk), jnp.float32)
        for t in range(n_codes // ktile):
            tile = mm_refs[c][t * ktile:(t + 1) * ktile, :]
            btile = bdotb[t * ktile:(t + 1) * ktile, :]
            d = (adota_c + tile) + btile
            mt = jnp.min(d, axis=0, keepdims=True)
            loc = jnp.min(
                jnp.where(d == mt, iotac, float(n_codes)),
                axis=0, keepdims=True)
            locg = loc + float(t * ktile)
            upd = mt < val
            val = jnp.where(upd, mt, val)
            idx = jnp.where(upd, locg, idx)
        o_ref[0, 0, lo:hi] = idx[0].astype(jnp.int32)


def kernel(X, weight):
    B, S, D = X.shape
    K = weight.shape[0]
    M = B * S
    x2 = X.reshape(M, D)

    BM = 4608
    CHUNK = 512
    nblk = M // BM

    out = pl.pallas_call(
        functools.partial(_nearest_idx_kernel, n_codes=K, bm=BM,
                          chunk=CHUNK, ktile=256),
        grid=(nblk,),
        in_specs=[
            pl.BlockSpec((BM, D), lambda i: (i, 0)),
            pl.BlockSpec((K, D), lambda i: (0, 0)),
        ],
        out_specs=pl.BlockSpec((1, 1, BM), lambda i: (i, 0, 0)),
        out_shape=jax.ShapeDtypeStruct((nblk, 1, BM), jnp.int32),
        scratch_shapes=(
            [pltpu.VMEM((K, 1), jnp.float32)]
            + [pltpu.VMEM((K, CHUNK), jnp.float32)
               for _ in range(BM // CHUNK)]),
        compiler_params=pltpu.CompilerParams(
            dimension_semantics=("arbitrary",)),
    )(x2, weight)
    return out.reshape(B, S)
